# baseline (speedup 1.0000x reference)
import functools

import jax
import jax.numpy as jnp
from jax import lax
from jax.experimental import pallas as pl
from jax.experimental.pallas import tpu as pltpu
from jax.experimental.pallas import tpu_sc as plsc

VOCAB = 1000
D_MODEL = 768
BATCH = 4096
SEQ = 50

_INFO = plsc.get_sparse_core_info()
_NC = _INFO.num_cores
_NS = _INFO.num_subcores
_NW = _NC * _NS
_B = BATCH * SEQ
_PER_W = _B // _NW
_CHUNK = 64
_NCHUNK = _PER_W // _CHUNK


@functools.partial(
    pl.kernel,
    mesh=plsc.VectorSubcoreMesh(core_axis_name="c", subcore_axis_name="s"),
    out_type=jax.ShapeDtypeStruct((_B, D_MODEL), jnp.float32),
    scratch_types=[
        pltpu.VMEM((_PER_W,), jnp.int32),
        pltpu.VMEM((_CHUNK, D_MODEL), jnp.float32),
        pltpu.VMEM((_CHUNK, D_MODEL), jnp.float32),
        pltpu.SemaphoreType.DMA,
        pltpu.SemaphoreType.DMA,
        pltpu.SemaphoreType.DMA,
        pltpu.SemaphoreType.DMA,
    ],
)
def _gather_kernel(table_hbm, idx_hbm, out_hbm, idx_v, rows0, rows1,
                   g0, g1, s0, s1):
    sid = lax.axis_index("s")
    wid = sid * _NC + lax.axis_index("c")
    base = wid * _PER_W
    bufs = (rows0, rows1)
    gsem = (g0, g1)
    ssem = (s0, s1)

    pltpu.sync_copy(idx_hbm.at[pl.ds(base, _PER_W)], idx_v)

    def gather(c, b):
        return pltpu.make_async_copy(
            table_hbm.at[idx_v.at[pl.ds(c * _CHUNK, _CHUNK)]], bufs[b], gsem[b]
        )

    def store(c, b):
        return pltpu.make_async_copy(
            bufs[b], out_hbm.at[pl.ds(base + c * _CHUNK, _CHUNK)], ssem[b]
        )

    gather(0, 0).start()
    gather(1, 1).start()

    def body(i, carry):
        for b in range(2):
            c = 2 * i + b
            gather(c, b).wait()
            store(c, b).start()
            store(c, b).wait()
            gather(c + 2, b).start()
        return carry

    lax.fori_loop(0, (_NCHUNK - 2) // 2, body, 0)

    for b in range(2):
        c = _NCHUNK - 2 + b
        gather(c, b).wait()
        store(c, b).start()
        store(c, b).wait()


def kernel(input_ids, embedding_table):
    flat_ids = input_ids.reshape(_B)
    out = _gather_kernel(embedding_table, flat_ids)
    return (out.reshape(BATCH, SEQ, D_MODEL),)


# speedup vs baseline: 3.0926x; 3.0926x over previous
import functools

import jax
import jax.numpy as jnp
from jax import lax
from jax.experimental import pallas as pl
from jax.experimental.pallas import tpu as pltpu
from jax.experimental.pallas import tpu_sc as plsc

VOCAB = 1000
D_MODEL = 768
BATCH = 4096
SEQ = 50

_INFO = plsc.get_sparse_core_info()
_NC = _INFO.num_cores
_NS = _INFO.num_subcores
_NW = _NC * _NS
_B = BATCH * SEQ
_PER_W = _B // _NW
_CHUNK = 64
_NCHUNK = _PER_W // _CHUNK


@functools.partial(
    pl.kernel,
    mesh=plsc.VectorSubcoreMesh(core_axis_name="c", subcore_axis_name="s"),
    out_type=jax.ShapeDtypeStruct((_B, D_MODEL), jnp.float32),
    scratch_types=[
        pltpu.VMEM((_PER_W,), jnp.int32),
        pltpu.VMEM((_CHUNK, D_MODEL), jnp.float32),
        pltpu.VMEM((_CHUNK, D_MODEL), jnp.float32),
        pltpu.SemaphoreType.DMA,
        pltpu.SemaphoreType.DMA,
        pltpu.SemaphoreType.DMA,
        pltpu.SemaphoreType.DMA,
    ],
)
def _gather_kernel(table_hbm, idx_hbm, out_hbm, idx_v, rows0, rows1,
                   g0, g1, s0, s1):
    wid = lax.axis_index("s") * _NC + lax.axis_index("c")
    base = wid * _PER_W
    bufs = (rows0, rows1)
    gsem = (g0, g1)
    ssem = (s0, s1)

    pltpu.sync_copy(idx_hbm.at[pl.ds(base, _PER_W)], idx_v)

    def gather(c, b):
        return pltpu.make_async_copy(
            table_hbm.at[idx_v.at[pl.ds(c * _CHUNK, _CHUNK)]], bufs[b], gsem[b]
        )

    def store(c, b):
        return pltpu.make_async_copy(
            bufs[b], out_hbm.at[pl.ds(base + c * _CHUNK, _CHUNK)], ssem[b]
        )

    gather(0, 0).start()
    gather(1, 1).start()

    def body(i, carry):
        for b in range(2):
            c = 2 * i + b
            gather(c, b).wait()
            store(c, b).start()
            store(c, b).wait()
            gather(c + 2, b).start()
        return carry

    lax.fori_loop(0, (_NCHUNK - 2) // 2, body, 0)

    for b in range(2):
        c = _NCHUNK - 2 + b
        gather(c, b).wait()
        store(c, b).start()
        store(c, b).wait()


def kernel(input_ids, embedding_table):
    flat_ids = input_ids.T.reshape(_B)
    out = _gather_kernel(embedding_table, flat_ids)
    return (out.reshape(SEQ, BATCH, D_MODEL).transpose(1, 0, 2),)


# speedup vs baseline: 3.1099x; 1.0056x over previous
import functools

import jax
import jax.numpy as jnp
from jax import lax
from jax.experimental import pallas as pl
from jax.experimental.pallas import tpu as pltpu
from jax.experimental.pallas import tpu_sc as plsc

VOCAB = 1000
D_MODEL = 768
BATCH = 4096
SEQ = 50

_INFO = plsc.get_sparse_core_info()
_NC = _INFO.num_cores
_NS = _INFO.num_subcores
_NW = _NC * _NS
_B = BATCH * SEQ
_PER_W = _B // _NW
_CHUNK = 40
_NCHUNK = _PER_W // _CHUNK


@functools.partial(
    pl.kernel,
    mesh=plsc.VectorSubcoreMesh(core_axis_name="c", subcore_axis_name="s"),
    out_type=jax.ShapeDtypeStruct((_B, D_MODEL), jnp.float32),
    scratch_types=[
        pltpu.VMEM((_PER_W,), jnp.int32),
        pltpu.VMEM((_CHUNK, D_MODEL), jnp.float32),
        pltpu.VMEM((_CHUNK, D_MODEL), jnp.float32),
        pltpu.VMEM((_CHUNK, D_MODEL), jnp.float32),
        pltpu.SemaphoreType.DMA,
        pltpu.SemaphoreType.DMA,
        pltpu.SemaphoreType.DMA,
        pltpu.SemaphoreType.DMA,
        pltpu.SemaphoreType.DMA,
        pltpu.SemaphoreType.DMA,
    ],
)
def _gather_kernel(table_hbm, idx_hbm, out_hbm, idx_v, rows0, rows1, rows2,
                   g0, g1, g2, s0, s1, s2):
    wid = lax.axis_index("s") * _NC + lax.axis_index("c")
    base = wid * _PER_W
    bufs = (rows0, rows1, rows2)
    gsem = (g0, g1, g2)
    ssem = (s0, s1, s2)

    pltpu.sync_copy(idx_hbm.at[pl.ds(base, _PER_W)], idx_v)

    def gather(c, b):
        return pltpu.make_async_copy(
            table_hbm.at[idx_v.at[pl.ds(c * _CHUNK, _CHUNK)]], bufs[b], gsem[b]
        )

    def store(c, b):
        return pltpu.make_async_copy(
            bufs[b], out_hbm.at[pl.ds(base + c * _CHUNK, _CHUNK)], ssem[b]
        )

    gather(0, 0).start()
    gather(1, 1).start()
    gather(2, 2).start()

    def body(i, carry):
        for b in range(3):
            c = 3 * i + b
            gather(c, b).wait()
            store(c, b).start()
            store(c, b).wait()
            gather(c + 3, b).start()
        return carry

    lax.fori_loop(0, (_NCHUNK - 4) // 3, body, 0)

    for c in range(_NCHUNK - 4, _NCHUNK):
        b = c % 3
        gather(c, b).wait()
        store(c, b).start()
        store(c, b).wait()
        if c + 3 < _NCHUNK:
            gather(c + 3, b).start()


def kernel(input_ids, embedding_table):
    flat_ids = input_ids.T.reshape(_B)
    out = _gather_kernel(embedding_table, flat_ids)
    return (out.reshape(SEQ, BATCH, D_MODEL).transpose(1, 0, 2),)


# speedup vs baseline: 3.1185x; 1.0028x over previous
import functools

import jax
import jax.numpy as jnp
from jax import lax
from jax.experimental import pallas as pl
from jax.experimental.pallas import tpu as pltpu
from jax.experimental.pallas import tpu_sc as plsc

VOCAB = 1000
D_MODEL = 768
BATCH = 4096
SEQ = 50

_INFO = plsc.get_sparse_core_info()
_NC = _INFO.num_cores
_NS = _INFO.num_subcores
_NW = _NC * _NS
_B = BATCH * SEQ
_PER_W = _B // _NW
_CHUNK = 32
_NCHUNK = _PER_W // _CHUNK


@functools.partial(
    pl.kernel,
    mesh=plsc.VectorSubcoreMesh(core_axis_name="c", subcore_axis_name="s"),
    out_type=jax.ShapeDtypeStruct((_B, D_MODEL), jnp.float32),
    scratch_types=[
        pltpu.VMEM((_PER_W,), jnp.int32),
        pltpu.VMEM((_CHUNK, D_MODEL), jnp.float32),
        pltpu.VMEM((_CHUNK, D_MODEL), jnp.float32),
        pltpu.VMEM((_CHUNK, D_MODEL), jnp.float32),
        pltpu.VMEM((_CHUNK, D_MODEL), jnp.float32),
        pltpu.SemaphoreType.DMA,
        pltpu.SemaphoreType.DMA,
        pltpu.SemaphoreType.DMA,
        pltpu.SemaphoreType.DMA,
        pltpu.SemaphoreType.DMA,
        pltpu.SemaphoreType.DMA,
        pltpu.SemaphoreType.DMA,
        pltpu.SemaphoreType.DMA,
    ],
)
def _gather_kernel(table_hbm, idx_hbm, out_hbm, idx_v, rows0, rows1, rows2,
                   rows3, g0, g1, g2, g3, s0, s1, s2, s3):
    wid = lax.axis_index("s") * _NC + lax.axis_index("c")
    base = wid * _PER_W
    bufs = (rows0, rows1, rows2, rows3)
    gsem = (g0, g1, g2, g3)
    ssem = (s0, s1, s2, s3)

    pltpu.sync_copy(idx_hbm.at[pl.ds(base, _PER_W)], idx_v)

    def gather(c, b):
        return pltpu.make_async_copy(
            table_hbm.at[idx_v.at[pl.ds(c * _CHUNK, _CHUNK)]], bufs[b], gsem[b]
        )

    def store(c, b):
        return pltpu.make_async_copy(
            bufs[b], out_hbm.at[pl.ds(base + c * _CHUNK, _CHUNK)], ssem[b]
        )

    for b in range(4):
        gather(b, b).start()

    def body(i, carry):
        for b in range(4):
            c = 4 * i + b
            gather(c, b).wait()
            store(c, b).start()
            store(c, b).wait()
            gather(c + 4, b).start()
        return carry

    lax.fori_loop(0, (_NCHUNK - 4) // 4, body, 0)

    for b in range(4):
        c = _NCHUNK - 4 + b
        gather(c, b).wait()
        store(c, b).start()
        store(c, b).wait()


def kernel(input_ids, embedding_table):
    flat_ids = input_ids.T.reshape(_B)
    out = _gather_kernel(embedding_table, flat_ids)
    return (out.reshape(SEQ, BATCH, D_MODEL).transpose(1, 0, 2),)


# speedup vs baseline: 3.1244x; 1.0019x over previous
import functools

import jax
import jax.numpy as jnp
from jax import lax
from jax.experimental import pallas as pl
from jax.experimental.pallas import tpu as pltpu
from jax.experimental.pallas import tpu_sc as plsc

VOCAB = 1000
D_MODEL = 768
BATCH = 4096
SEQ = 50

_INFO = plsc.get_sparse_core_info()
_NC = _INFO.num_cores
_NS = _INFO.num_subcores
_NW = _NC * _NS
_B = BATCH * SEQ
_PER_W = _B // _NW
_CHUNK = 16
_NCHUNK = _PER_W // _CHUNK


@functools.partial(
    pl.kernel,
    mesh=plsc.VectorSubcoreMesh(core_axis_name="c", subcore_axis_name="s"),
    out_type=jax.ShapeDtypeStruct((_B, D_MODEL), jnp.float32),
    scratch_types=[
        pltpu.VMEM((_PER_W,), jnp.int32),
        pltpu.VMEM((_CHUNK, D_MODEL), jnp.float32),
        pltpu.VMEM((_CHUNK, D_MODEL), jnp.float32),
        pltpu.VMEM((_CHUNK, D_MODEL), jnp.float32),
        pltpu.VMEM((_CHUNK, D_MODEL), jnp.float32),
        pltpu.VMEM((_CHUNK, D_MODEL), jnp.float32),
        pltpu.VMEM((_CHUNK, D_MODEL), jnp.float32),
        pltpu.VMEM((_CHUNK, D_MODEL), jnp.float32),
        pltpu.VMEM((_CHUNK, D_MODEL), jnp.float32),
        pltpu.SemaphoreType.DMA,
        pltpu.SemaphoreType.DMA,
        pltpu.SemaphoreType.DMA,
        pltpu.SemaphoreType.DMA,
        pltpu.SemaphoreType.DMA,
        pltpu.SemaphoreType.DMA,
        pltpu.SemaphoreType.DMA,
        pltpu.SemaphoreType.DMA,
        pltpu.SemaphoreType.DMA,
        pltpu.SemaphoreType.DMA,
        pltpu.SemaphoreType.DMA,
        pltpu.SemaphoreType.DMA,
        pltpu.SemaphoreType.DMA,
        pltpu.SemaphoreType.DMA,
        pltpu.SemaphoreType.DMA,
        pltpu.SemaphoreType.DMA,
    ],
)
def _gather_kernel(table_hbm, idx_hbm, out_hbm, idx_v,
                   r0, r1, r2, r3, r4, r5, r6, r7,
                   g0, g1, g2, g3, g4, g5, g6, g7,
                   s0, s1, s2, s3, s4, s5, s6, s7):
    wid = lax.axis_index("s") * _NC + lax.axis_index("c")
    base = wid * _PER_W
    bufs = (r0, r1, r2, r3, r4, r5, r6, r7)
    gsem = (g0, g1, g2, g3, g4, g5, g6, g7)
    ssem = (s0, s1, s2, s3, s4, s5, s6, s7)

    pltpu.sync_copy(idx_hbm.at[pl.ds(base, _PER_W)], idx_v)

    def gather(c, b):
        return pltpu.make_async_copy(
            table_hbm.at[idx_v.at[pl.ds(c * _CHUNK, _CHUNK)]], bufs[b], gsem[b]
        )

    def store(c, b):
        return pltpu.make_async_copy(
            bufs[b], out_hbm.at[pl.ds(base + c * _CHUNK, _CHUNK)], ssem[b]
        )

    for b in range(8):
        gather(b, b).start()

    def body(i, carry):
        for b in range(8):
            c = 8 * i + b
            gather(c, b).wait()
            store(c, b).start()
            store(c, b).wait()
            gather(c + 8, b).start()
        return carry

    lax.fori_loop(0, (_NCHUNK - 8) // 8, body, 0)

    for b in range(8):
        c = _NCHUNK - 8 + b
        gather(c, b).wait()
        store(c, b).start()
        store(c, b).wait()


def kernel(input_ids, embedding_table):
    flat_ids = input_ids.T.reshape(_B)
    out = _gather_kernel(embedding_table, flat_ids)
    return (out.reshape(SEQ, BATCH, D_MODEL).transpose(1, 0, 2),)
